# Initial kernel scaffold; baseline (speedup 1.0000x reference)
import jax
import jax.numpy as jnp
from jax.experimental import pallas as pl


def kernel(mus, random_effects, annotators, annotations, confidences):
    raise NotImplementedError("write your pallas kernel here")


# speedup vs baseline: 2.3193x; 2.3193x over previous
import functools

import jax
import jax.numpy as jnp
from jax import lax
from jax.experimental import pallas as pl
from jax.experimental.pallas import tpu as pltpu
from jax.experimental.pallas import tpu_sc as plsc


def _sc_gather(table, idx3, n, d):
    nw, nch, ch = idx3.shape
    b_per_w = nch * ch
    mesh = plsc.VectorSubcoreMesh(core_axis_name="c", subcore_axis_name="s")

    @functools.partial(
        pl.kernel,
        mesh=mesh,
        compiler_params=pltpu.CompilerParams(use_tc_tiling_on_sc=False),
        out_type=jax.ShapeDtypeStruct((n, d), jnp.float32),
        scratch_types=[
            pltpu.VMEM((nch, ch), jnp.int32),
            pltpu.VMEM((b_per_w, d), jnp.float32),
            pltpu.SemaphoreType.DMA,
        ],
    )
    def k(table_hbm, idx_hbm, out_hbm, idx_v, rows_v, sem):
        wid = lax.axis_index("s") * mesh.num_cores + lax.axis_index("c")
        base = wid * b_per_w
        pltpu.sync_copy(idx_hbm.at[wid], idx_v)
        copies = [
            pltpu.async_copy(
                table_hbm.at[idx_v.at[j]], rows_v.at[pl.ds(j * ch, ch)], sem
            )
            for j in range(nch)
        ]
        for c in copies:
            c.wait()
        pltpu.sync_copy(rows_v, out_hbm.at[pl.ds(base, b_per_w)])

    return k(table, idx3)


def _tc_body(mus_ref, r_ref, ann_ref, conf_ref, out_ref):
    mu = mus_ref[...]
    r = r_ref[...]
    a = ann_ref[...]
    cf = conf_ref[...]
    blk, dd = r.shape
    iota = lax.broadcasted_iota(jnp.int32, (blk, dd), 1)
    onehot = (iota == a).astype(jnp.float32)
    r_pick = jnp.sum(r * onehot, axis=1, keepdims=True)
    dn = (((1,), (1,)), ((), ()))
    mu_pick = lax.dot_general(onehot, mu, dn,
                              preferred_element_type=jnp.float32)
    p = lax.dot_general(jnp.exp(r), jnp.exp(mu), dn,
                        preferred_element_type=jnp.float32)
    out_ref[...] = cf * (mu_pick + r_pick - jnp.log(p))


def _tc_compute(r, mus, ann2, conf2, blk):
    n, d = r.shape
    c = mus.shape[0]
    grid = n // blk
    return pl.pallas_call(
        _tc_body,
        grid=(grid,),
        in_specs=[
            pl.BlockSpec((c, d), lambda i: (0, 0)),
            pl.BlockSpec((blk, d), lambda i: (i, 0)),
            pl.BlockSpec((blk, 1), lambda i: (i, 0)),
            pl.BlockSpec((blk, 1), lambda i: (i, 0)),
        ],
        out_specs=pl.BlockSpec((blk, c), lambda i: (i, 0)),
        out_shape=jax.ShapeDtypeStruct((n, c), jnp.float32),
    )(mus, r, ann2, conf2)


def kernel(mus, random_effects, annotators, annotations, confidences):
    n = annotators.shape[0]
    d = random_effects.shape[1]
    nw = 32
    ch = 128
    nch = n // (nw * ch)
    idx3 = annotators.reshape(nw, nch, ch)
    r = _sc_gather(random_effects, idx3, n, d)
    ann2 = annotations.reshape(n, 1)
    conf2 = confidences.reshape(n, 1)
    return _tc_compute(r, mus, ann2, conf2, blk=2048)
